# Initial kernel scaffold; baseline (speedup 1.0000x reference)
import jax
import jax.numpy as jnp
from jax.experimental import pallas as pl


def kernel(h, edge_index1, edge_index2, W1, b1, p, W2, b2):
    raise NotImplementedError("write your pallas kernel here")


# speedup vs baseline: 4.3268x; 4.3268x over previous
import functools

import jax
import jax.numpy as jnp
from jax import lax
from jax.experimental import pallas as pl
from jax.experimental.pallas import tpu as pltpu
from jax.experimental.pallas import tpu_sc as plsc

N = 10000
E = 320000
D = 128
NPAD = 10240

NC = 2
NS = 16
NW = NC * NS

CHUNK = 128
NCHUNKS = E // CHUNK
BASE_K = NCHUNKS // NW
EXTRA = NCHUNKS - BASE_K * NW

ZCH = NPAD // NS

_MESH = dict(mesh=plsc.VectorSubcoreMesh(core_axis_name="c", subcore_axis_name="s"))


def _worker_id():
    c = lax.axis_index("c")
    s = lax.axis_index("s")
    w = c * NS + s
    nk = jnp.where(w < EXTRA, BASE_K + 1, BASE_K)
    return c, s, w, nk


_cnt_out = jax.ShapeDtypeStruct((NC, NPAD), jnp.float32)


@functools.partial(
    pl.kernel,
    out_type=(_cnt_out,) * 4,
    scratch_types=[
        pltpu.VMEM((CHUNK,), jnp.int32),
        pltpu.VMEM((CHUNK,), jnp.float32),
        pltpu.VMEM_SHARED((NPAD,), jnp.float32),
        pltpu.VMEM_SHARED((NPAD,), jnp.float32),
        pltpu.VMEM_SHARED((NPAD,), jnp.float32),
        pltpu.VMEM_SHARED((NPAD,), jnp.float32),
    ],
    **_MESH,
)
def _sc_degrees(idx_hbm, z_hbm, c0, c1, c2, c3,
                idx_v, ones_v, a0, a1, a2, a3):
    c, s, w, nk = _worker_id()
    accs = [a0, a1, a2, a3]
    outs = [c0, c1, c2, c3]
    for i in range(CHUNK // 16):
        ones_v[pl.ds(i * 16, 16)] = jnp.full((16,), 1.0, jnp.float32)

    zb = s * ZCH
    for a in accs:
        pltpu.sync_copy(z_hbm.at[pl.ds(zb, ZCH)], a.at[pl.ds(zb, ZCH)])
    plsc.subcore_barrier()

    for a in range(4):
        acc = accs[a]

        def body(k, _, a=a, acc=acc):
            base = (w + k * NW) * CHUNK
            pltpu.sync_copy(idx_hbm.at[a].at[pl.ds(base, CHUNK)], idx_v)
            pltpu.sync_copy(ones_v, acc.at[idx_v], add=True)
            return ()

        lax.fori_loop(0, nk, body, ())

    plsc.subcore_barrier()
    for a in range(4):
        pltpu.sync_copy(accs[a].at[pl.ds(zb, ZCH)],
                        outs[a].at[c].at[pl.ds(zb, ZCH)])


@functools.partial(
    pl.kernel,
    out_type=jax.ShapeDtypeStruct((NC, NPAD, D), jnp.float32),
    scratch_types=[
        pltpu.VMEM((CHUNK,), jnp.int32),
        pltpu.VMEM((CHUNK,), jnp.int32),
        pltpu.VMEM((CHUNK, D), jnp.float32),
        pltpu.VMEM_SHARED((NPAD, D), jnp.float32),
        pltpu.SemaphoreType.DMA,
    ],
    **_MESH,
)
def _sc_edge_agg(feat_hbm, src_hbm, dst_hbm, z_hbm, out_hbm,
                 sidx_v, didx_v, rows_v, acc, sem):
    c, s, w, nk = _worker_id()
    rb = s * ZCH
    pltpu.sync_copy(z_hbm.at[pl.ds(rb, ZCH)], acc.at[pl.ds(rb, ZCH)])
    plsc.subcore_barrier()

    def body(k, _):
        base = (w + k * NW) * CHUNK
        pltpu.sync_copy(src_hbm.at[pl.ds(base, CHUNK)], sidx_v)
        pltpu.sync_copy(dst_hbm.at[pl.ds(base, CHUNK)], didx_v)
        pltpu.async_copy(feat_hbm.at[sidx_v], rows_v, sem).wait()
        pltpu.sync_copy(rows_v, acc.at[didx_v], add=True)
        return ()

    lax.fori_loop(0, nk, body, ())

    plsc.subcore_barrier()
    pltpu.sync_copy(acc.at[pl.ds(rb, ZCH)],
                    out_hbm.at[c].at[pl.ds(rb, ZCH)])


def _tc1_body(h_ref, n_ref, w_ref, o_ref):
    x = h_ref[...] * n_ref[:, 0:1]
    o_ref[...] = jnp.dot(x, w_ref[...], preferred_element_type=jnp.float32)


def _tc2_body(p_ref, n_ref, b_ref, pc_ref, w_ref, o_ref):
    agg = p_ref[0, :N] + p_ref[1, :N]
    x = jnp.maximum(agg * n_ref[:, 1:2] + b_ref[...], 0.0)
    x = x * jnp.clip(pc_ref[...], 0.0, 1.0)
    o_ref[...] = jnp.dot(x * n_ref[:, 2:3], w_ref[...],
                         preferred_element_type=jnp.float32)


def _tc3_body(p_ref, n_ref, b_ref, o_ref):
    o_ref[...] = (p_ref[0, :N] + p_ref[1, :N]) * n_ref[:, 3:4] + b_ref[...]


_f32 = jnp.float32


def _tc1(h, norms, W1):
    return pl.pallas_call(
        _tc1_body, out_shape=jax.ShapeDtypeStruct((N, D), _f32))(h, norms, W1)


def _tc2(part1, norms, b1, p, W2):
    return pl.pallas_call(
        _tc2_body, out_shape=jax.ShapeDtypeStruct((N, D), _f32))(
            part1, norms, b1.reshape(1, D), p.reshape(1, D), W2)


def _tc3(part2, norms, b2):
    return pl.pallas_call(
        _tc3_body, out_shape=jax.ShapeDtypeStruct((N, D), _f32))(
            part2, norms, b2.reshape(1, D))


def kernel(h, edge_index1, edge_index2, W1, b1, p, W2, b2):
    src1 = edge_index1[0].astype(jnp.int32)
    dst1 = edge_index1[1].astype(jnp.int32)
    src2 = edge_index2[0].astype(jnp.int32)
    dst2 = edge_index2[1].astype(jnp.int32)

    idx_all = jnp.stack([src1, dst1, src2, dst2])
    zeros_n = jnp.zeros((NPAD,), jnp.float32)
    zeros_nd = jnp.zeros((NPAD, D), jnp.float32)

    cnts = _sc_degrees(idx_all, zeros_n)
    norms = jnp.stack(
        [lax.rsqrt(jnp.clip(ca[0, :N] + ca[1, :N], 1.0, None)) for ca in cnts],
        axis=1)

    feat1 = _tc1(h, norms, W1)
    part1 = _sc_edge_agg(feat1, src1, dst1, zeros_nd)
    feat2 = _tc2(part1, norms, b1, p, W2)
    part2 = _sc_edge_agg(feat2, src2, dst2, zeros_nd)
    return _tc3(part2, norms, b2)
